# baseline (speedup 1.0000x reference)
import functools

import jax
import jax.numpy as jnp
from jax import lax
from jax.experimental import pallas as pl
from jax.experimental.pallas import tpu as pltpu
from jax.experimental.pallas import tpu_sc as plsc

D = 64
B = 16384 * 26
NPAIR = 500000
NC, NS = 2, 16
NW = NC * NS
CHUNK = 128
ROWS_PER_W = B // NW
NCHUNK = ROWS_PER_W // CHUNK
NBUF = 4
L = 16

_mesh = plsc.VectorSubcoreMesh(core_axis_name="c", subcore_axis_name="s")


@functools.partial(
    pl.kernel,
    out_type=jax.ShapeDtypeStruct((B, 2 * D), jnp.float32),
    mesh=_mesh,
    scratch_types=(
        [pltpu.VMEM((ROWS_PER_W,), jnp.int32)]
        + [pltpu.VMEM((CHUNK,), jnp.int32) for _ in range(NBUF)]
        + [pltpu.VMEM((CHUNK, 2 * D), jnp.float32) for _ in range(NBUF)]
        + [pltpu.SemaphoreType.DMA for _ in range(NBUF)]
    ),
)
def _gather_pairs(ids_hbm, table_hbm, out_hbm, idx_v, *scratch):
    pids = scratch[:NBUF]
    bufs = scratch[NBUF:2 * NBUF]
    sems = scratch[2 * NBUF:]
    wid = lax.axis_index("s") * NC + lax.axis_index("c")
    tok0 = wid * ROWS_PER_W

    pltpu.sync_copy(ids_hbm.at[pl.ds(tok0, ROWS_PER_W)], idx_v)

    def fill_pid(j, b):
        def body(v, _):
            pids[b][pl.ds(v * L, L)] = (
                idx_v[pl.ds(j * CHUNK + v * L, L)] >> 1
            )
            return _
        lax.fori_loop(0, CHUNK // L, body, 0)

    for b in range(NBUF):
        fill_pid(b, b)
        pltpu.async_copy(table_hbm.at[pids[b]], bufs[b], sems[b])

    def outer(g, carry):
        base = g * NBUF
        for b in range(NBUF):
            j = base + b
            pltpu.make_async_copy(table_hbm.at[pids[b]], bufs[b], sems[b]).wait()
            pltpu.sync_copy(bufs[b], out_hbm.at[pl.ds((tok0 + j * CHUNK), CHUNK)])
            fill_pid(j + NBUF, b)
            pltpu.async_copy(table_hbm.at[pids[b]], bufs[b], sems[b])
        return carry

    lax.fori_loop(0, (NCHUNK - NBUF) // NBUF, outer, 0)

    for b in range(NBUF):
        j = NCHUNK - NBUF + b
        pltpu.make_async_copy(table_hbm.at[pids[b]], bufs[b], sems[b]).wait()
        pltpu.sync_copy(bufs[b], out_hbm.at[pl.ds((tok0 + j * CHUNK), CHUNK)])


def kernel(token_ids, weight):
    flat = token_ids.reshape(-1).astype(jnp.int32)
    wpair = weight.reshape(NPAIR, 2 * D)
    raw = _gather_pairs(flat, wpair)
    g = raw.reshape(16384, 26, 2, D)
    par = (token_ids.astype(jnp.int32) & 1)[..., None] == 1
    return jnp.where(par, g[:, :, 1, :], g[:, :, 0, :])


# speedup vs baseline: 2.2493x; 2.2493x over previous
import functools

import jax
import jax.numpy as jnp
from jax import lax
from jax.experimental import pallas as pl
from jax.experimental.pallas import tpu as pltpu
from jax.experimental.pallas import tpu_sc as plsc

D = 64
NSEQ = 26
NB = 16384
NC, NS, L = 2, 16, 16
NW = NC * NS
IB = NB // 128
NUNIT = NSEQ * IB
UNITS_PER_W = NUNIT // NW

_mesh = plsc.VectorSubcoreMesh(core_axis_name="c", subcore_axis_name="s")


@functools.partial(
    pl.kernel,
    out_type=jax.ShapeDtypeStruct((NSEQ, D, NB), jnp.float32),
    mesh=_mesh,
    scratch_types=(
        [pltpu.VMEM((128,), jnp.int32) for _ in range(2)]
        + [pltpu.VMEM((128, D), jnp.float32) for _ in range(2)]
        + [pltpu.VMEM((D, 128), jnp.float32) for _ in range(2)]
        + [pltpu.SemaphoreType.DMA, pltpu.SemaphoreType.DMA,
           pltpu.SemaphoreType.DMA, pltpu.SemaphoreType.DMA]
    ),
    compiler_params=pltpu.CompilerParams(
        use_tc_tiling_on_sc=False, needs_layout_passes=False),
)
def _gather_t(ids_hbm, table_hbm, out_hbm,
              iv0, iv1, gb0, gb1, ob0, ob1, gs0, gs1, os0, os1):
    ivs = (iv0, iv1)
    gbufs, obufs = (gb0, gb1), (ob0, ob1)
    gsems, osems = (gs0, gs1), (os0, os1)
    wid = lax.axis_index("s") * NC + lax.axis_index("c")
    u0 = wid * UNITS_PER_W

    def prep(k, b):
        u = u0 + k
        s = u // IB
        ib = lax.rem(u, IB)
        pltpu.sync_copy(ids_hbm.at[s, pl.ds(ib * 128, 128)], ivs[b])
        pltpu.async_copy(table_hbm.at[ivs[b]], gbufs[b], gsems[b])

    def transpose_unit(b):
        def body(g, carry):
            rows = lax.iota(jnp.int32, L) + g * L
            for d in range(D):
                obufs[b][d, pl.ds(g * L, L)] = plsc.load_gather(
                    gbufs[b], [rows, jnp.full((L,), d, jnp.int32)])
            return carry
        lax.fori_loop(0, 128 // L, body, 0)

    prep(0, 0)
    prep(1, 1)

    def outer(k, carry):
        def work(bb):
            u = u0 + k
            s = u // IB
            ib = lax.rem(u, IB)
            pltpu.make_async_copy(
                table_hbm.at[ivs[bb]], gbufs[bb], gsems[bb]).wait()

            @pl.when(k >= 2)
            def _():
                pltpu.make_async_copy(
                    obufs[bb], out_hbm.at[0, :, pl.ds(0, 128)], osems[bb]).wait()
            transpose_unit(bb)
            pltpu.async_copy(
                obufs[bb], out_hbm.at[s, :, pl.ds(ib * 128, 128)], osems[bb])

            @pl.when(k + 2 < UNITS_PER_W)
            def _():
                prep(k + 2, bb)

        lax.cond(lax.rem(k, 2) == 0, lambda: work(0), lambda: work(1))
        return carry

    lax.fori_loop(0, UNITS_PER_W, outer, 0)
    for b in range(2):
        pltpu.make_async_copy(
            obufs[b], out_hbm.at[0, :, pl.ds(0, 128)], osems[b]).wait()


def kernel(token_ids, weight):
    ids_t = token_ids.T.astype(jnp.int32)
    vt = _gather_t(ids_t, weight)
    return vt.transpose(2, 0, 1)


# speedup vs baseline: 3.2911x; 1.4631x over previous
import functools

import jax
import jax.numpy as jnp
from jax import lax
from jax.experimental import pallas as pl
from jax.experimental.pallas import tpu as pltpu
from jax.experimental.pallas import tpu_sc as plsc

D = 64
B = 16384 * 26
NC, NS = 2, 16
NW = NC * NS
CHUNK = 128
ROWS_PER_W = B // NW
NCHUNK = ROWS_PER_W // CHUNK
NBUF = 8

_mesh = plsc.VectorSubcoreMesh(core_axis_name="c", subcore_axis_name="s")


@functools.partial(
    pl.kernel,
    out_type=jax.ShapeDtypeStruct((B, D), jnp.float32),
    mesh=_mesh,
    scratch_types=(
        [pltpu.VMEM((NCHUNK, CHUNK), jnp.int32)]
        + [pltpu.VMEM((CHUNK, D), jnp.float32) for _ in range(NBUF)]
        + [pltpu.SemaphoreType.DMA for _ in range(NBUF)]
    ),
    compiler_params=pltpu.CompilerParams(use_tc_tiling_on_sc=False),
)
def _embed_sc(idx_hbm, table_hbm, out_hbm, idx_v, *bufs_and_sems):
    bufs = bufs_and_sems[:NBUF]
    sems = bufs_and_sems[NBUF:]
    wid = lax.axis_index("s") * NC + lax.axis_index("c")
    chunk0 = wid * NCHUNK

    pltpu.sync_copy(idx_hbm.at[pl.ds(chunk0, NCHUNK)], idx_v)

    for b in range(NBUF):
        pltpu.async_copy(table_hbm.at[idx_v.at[b]], bufs[b], sems[b])

    def outer(g, carry):
        base = g * NBUF
        for b in range(NBUF):
            j = base + b
            pltpu.make_async_copy(table_hbm.at[idx_v.at[j]], bufs[b], sems[b]).wait()
            pltpu.sync_copy(bufs[b], out_hbm.at[pl.ds((chunk0 + j) * CHUNK, CHUNK)])
            pltpu.async_copy(table_hbm.at[idx_v.at[j + NBUF]], bufs[b], sems[b])
        return carry

    lax.fori_loop(0, (NCHUNK - NBUF) // NBUF, outer, 0)

    for b in range(NBUF):
        j = NCHUNK - NBUF + b
        pltpu.make_async_copy(table_hbm.at[idx_v.at[j]], bufs[b], sems[b]).wait()
        pltpu.sync_copy(bufs[b], out_hbm.at[pl.ds((chunk0 + j) * CHUNK, CHUNK)])


def kernel(token_ids, weight):
    flat = token_ids.reshape(B // CHUNK, CHUNK).astype(jnp.int32)
    out = _embed_sc(flat, weight)
    return out.reshape(token_ids.shape + (weight.shape[1],))


# speedup vs baseline: 3.3047x; 1.0041x over previous
import functools

import jax
import jax.numpy as jnp
from jax import lax
from jax.experimental import pallas as pl
from jax.experimental.pallas import tpu as pltpu
from jax.experimental.pallas import tpu_sc as plsc

D = 64
B = 16384 * 26
NC, NS = 2, 16
NW = NC * NS
CHUNK = 128
ROWS_PER_W = B // NW
NCHUNK = ROWS_PER_W // CHUNK
NBUF = 4

_mesh = plsc.VectorSubcoreMesh(core_axis_name="c", subcore_axis_name="s")


@functools.partial(
    pl.kernel,
    out_type=jax.ShapeDtypeStruct((B, D), jnp.float32),
    mesh=_mesh,
    scratch_types=(
        [pltpu.VMEM((NCHUNK, CHUNK), jnp.int32)]
        + [pltpu.VMEM((CHUNK, D), jnp.float32) for _ in range(NBUF)]
        + [pltpu.SemaphoreType.DMA for _ in range(NBUF)]
    ),
    compiler_params=pltpu.CompilerParams(use_tc_tiling_on_sc=False),
)
def _embed_sc(idx_hbm, table_hbm, out_hbm, idx_v, *bufs_and_sems):
    bufs = bufs_and_sems[:NBUF]
    sems = bufs_and_sems[NBUF:]
    wid = lax.axis_index("s") * NC + lax.axis_index("c")
    chunk0 = wid * NCHUNK

    pltpu.sync_copy(idx_hbm.at[pl.ds(chunk0, NCHUNK)], idx_v)

    for b in range(NBUF):
        pltpu.async_copy(table_hbm.at[idx_v.at[b]], bufs[b], sems[b])

    def outer(g, carry):
        base = g * NBUF
        for b in range(NBUF):
            j = base + b
            pltpu.make_async_copy(table_hbm.at[idx_v.at[j]], bufs[b], sems[b]).wait()
            pltpu.sync_copy(bufs[b], out_hbm.at[pl.ds((chunk0 + j) * CHUNK, CHUNK)])
            pltpu.async_copy(table_hbm.at[idx_v.at[j + NBUF]], bufs[b], sems[b])
        return carry

    lax.fori_loop(0, (NCHUNK - NBUF) // NBUF, outer, 0)

    for b in range(NBUF):
        j = NCHUNK - NBUF + b
        pltpu.make_async_copy(table_hbm.at[idx_v.at[j]], bufs[b], sems[b]).wait()
        pltpu.sync_copy(bufs[b], out_hbm.at[pl.ds((chunk0 + j) * CHUNK, CHUNK)])


def kernel(token_ids, weight):
    flat = token_ids.reshape(B // CHUNK, CHUNK).astype(jnp.int32)
    out = _embed_sc(flat, weight)
    return out.reshape(token_ids.shape + (weight.shape[1],))
